# baseline (speedup 1.0000x reference)
import functools
import math

import jax
import jax.numpy as jnp
from jax import lax
from jax.experimental import pallas as pl
from jax.experimental.pallas import tpu as pltpu
from jax.experimental.pallas import tpu_sc as plsc

N_NODES = 10000
N_PAD = 10240
N_EDGES = 320000
HIDDEN = 128
N_LAYERS = 4
INV_SQRT_DEG = 1.0 / math.sqrt(32.0)

_NC = 2
_NS = 16
_NW = _NC * _NS
_EPW = N_EDGES // _NW
_CHUNK = 40
_NCHUNK = _EPW // _CHUNK
_RPT = N_PAD // _NS


def _edge_body(a_hbm, b_hbm, idx_hbm, out_hbm,
               ib0, ib1, ib2, ib3, av0, av1, bv0, bv1, mv0, mv1, acc,
               si0, si1, si2, si3, sga0, sga1, sgb0, sgb1, ssc0, ssc1):
    c = lax.axis_index("c")
    s = lax.axis_index("s")
    wid = c * _NS + s
    ib = (ib0, ib1, ib2, ib3)
    sidx = (si0, si1, si2, si3)
    av = (av0, av1)
    bv = (bv0, bv1)
    mv = (mv0, mv1)
    sga = (sga0, sga1)
    sgb = (sgb0, sgb1)
    ssc = (ssc0, ssc1)

    for j in range(2):
        pltpu.async_copy(idx_hbm.at[wid, j], ib[j], sidx[j])

    zero16 = jnp.zeros((16,), jnp.float32)

    @plsc.parallel_loop(0, _CHUNK, step=1, unroll=2)
    def _zrow(i):
        for j in range(HIDDEN // 16):
            mv0[i, pl.ds(j * 16, 16)] = zero16
    for r in range(_RPT // _CHUNK):
        pltpu.sync_copy(mv0, acc.at[pl.ds(s * _RPT + r * _CHUNK, _CHUNK)])

    for j in range(2):
        pltpu.make_async_copy(idx_hbm.at[wid, j], ib[j], sidx[j]).wait()
        pltpu.async_copy(a_hbm.at[ib[j].at[0]], av[j], sga[j])
        pltpu.async_copy(b_hbm.at[ib[j].at[1]], bv[j], sgb[j])
    plsc.subcore_barrier()

    def _outer(g, carry):
        for b in range(4):
            k = 4 * g + b
            b2 = b % 2

            @pl.when(k < _NCHUNK)
            def _chunk():
                pltpu.make_async_copy(a_hbm.at[ib[b].at[0]], av[b2], sga[b2]).wait()
                pltpu.make_async_copy(b_hbm.at[ib[b].at[1]], bv[b2], sgb[b2]).wait()

                @pl.when(k >= 2)
                def _drain_prev_scatter():
                    pltpu.make_async_copy(
                        mv[b2], acc.at[ib[b].at[0]], ssc[b2]).wait()

                @pl.when(k + 2 < _NCHUNK)
                def _next_idx():
                    pltpu.async_copy(idx_hbm.at[wid, k + 2],
                                     ib[(b + 2) % 4], sidx[(b + 2) % 4])

                @plsc.parallel_loop(0, _CHUNK, step=1, unroll=2)
                def _row(i):
                    for j in range(HIDDEN // 16):
                        sl = pl.ds(j * 16, 16)
                        v = av[b2][i, sl] + bv[b2][i, sl]
                        mv[b2][i, sl] = v / (1.0 + jnp.exp(-v))

                @pl.when(k + 2 < _NCHUNK)
                def _next_gather():
                    pltpu.make_async_copy(idx_hbm.at[wid, k + 2],
                                          ib[(b + 2) % 4],
                                          sidx[(b + 2) % 4]).wait()
                    pltpu.async_copy(a_hbm.at[ib[(b + 2) % 4].at[0]],
                                     av[b2], sga[b2])
                    pltpu.async_copy(b_hbm.at[ib[(b + 2) % 4].at[1]],
                                     bv[b2], sgb[b2])

                pltpu.async_copy(mv[b2], acc.at[ib[b].at[0]], ssc[b2], add=True)

        return carry

    lax.fori_loop(0, (_NCHUNK + 3) // 4, _outer, 0)

    for b2 in range(2):
        pltpu.make_async_copy(mv[b2], acc.at[ib[0].at[0]], ssc[b2]).wait()

    plsc.subcore_barrier()
    off = s * _RPT
    pltpu.sync_copy(acc.at[pl.ds(off, _RPT)], out_hbm.at[c, pl.ds(off, _RPT)])


@functools.lru_cache(maxsize=None)
def _make_edge_pass():
    return functools.partial(
        pl.kernel,
        out_type=jax.ShapeDtypeStruct((_NC, N_PAD, HIDDEN), jnp.float32),
        mesh=plsc.VectorSubcoreMesh(core_axis_name="c", subcore_axis_name="s",
                                    num_cores=_NC, num_subcores=_NS),
        scratch_types=(
            [pltpu.VMEM((2, _CHUNK), jnp.int32) for _ in range(4)]
            + [pltpu.VMEM((_CHUNK, HIDDEN), jnp.float32) for _ in range(6)]
            + [pltpu.VMEM_SHARED((N_PAD, HIDDEN), jnp.float32)]
            + [pltpu.SemaphoreType.DMA for _ in range(10)]
        ),
    )(_edge_body)


_ROWS = 1280
_GRID = N_PAD // _ROWS


def _embed_body(f_ref, w_ref, b_ref, wc_ref, bc_ref, x_ref, a_ref, b2_ref):
    x = jnp.dot(f_ref[...], w_ref[...], preferred_element_type=jnp.float32)
    x = jax.nn.silu(x + b_ref[...])
    x_ref[...] = x
    ab = jnp.dot(x, wc_ref[...], preferred_element_type=jnp.float32) + bc_ref[...]
    a_ref[...] = ab[:, :HIDDEN]
    b2_ref[...] = ab[:, HIDDEN:]


def _embed_call(f, w, b, wc, bc):
    return pl.pallas_call(
        _embed_body,
        grid=(_GRID,),
        in_specs=[
            pl.BlockSpec((_ROWS, HIDDEN), lambda i: (i, 0)),
            pl.BlockSpec((HIDDEN, HIDDEN), lambda i: (0, 0)),
            pl.BlockSpec((1, HIDDEN), lambda i: (0, 0)),
            pl.BlockSpec((HIDDEN, 2 * HIDDEN), lambda i: (0, 0)),
            pl.BlockSpec((1, 2 * HIDDEN), lambda i: (0, 0)),
        ],
        out_specs=[
            pl.BlockSpec((_ROWS, HIDDEN), lambda i: (i, 0)),
            pl.BlockSpec((_ROWS, HIDDEN), lambda i: (i, 0)),
            pl.BlockSpec((_ROWS, HIDDEN), lambda i: (i, 0)),
        ],
        out_shape=[
            jax.ShapeDtypeStruct((N_PAD, HIDDEN), jnp.float32),
            jax.ShapeDtypeStruct((N_PAD, HIDDEN), jnp.float32),
            jax.ShapeDtypeStruct((N_PAD, HIDDEN), jnp.float32),
        ],
    )(f, w, b, wc, bc)


def _mid_body(x_ref, p_ref, u_ref, bu_ref, wc_ref, bc_ref, x_out, a_ref, b2_ref):
    x = x_ref[...]
    m = (p_ref[0] + p_ref[1]) * INV_SQRT_DEG
    h = jnp.dot(x + m, u_ref[...], preferred_element_type=jnp.float32)
    xn = x + jax.nn.silu(h + bu_ref[...])
    x_out[...] = xn
    ab = jnp.dot(xn, wc_ref[...], preferred_element_type=jnp.float32) + bc_ref[...]
    a_ref[...] = ab[:, :HIDDEN]
    b2_ref[...] = ab[:, HIDDEN:]


def _mid_call(x, p, u, bu, wc, bc):
    return pl.pallas_call(
        _mid_body,
        grid=(_GRID,),
        in_specs=[
            pl.BlockSpec((_ROWS, HIDDEN), lambda i: (i, 0)),
            pl.BlockSpec((_NC, _ROWS, HIDDEN), lambda i: (0, i, 0)),
            pl.BlockSpec((HIDDEN, HIDDEN), lambda i: (0, 0)),
            pl.BlockSpec((1, HIDDEN), lambda i: (0, 0)),
            pl.BlockSpec((HIDDEN, 2 * HIDDEN), lambda i: (0, 0)),
            pl.BlockSpec((1, 2 * HIDDEN), lambda i: (0, 0)),
        ],
        out_specs=[
            pl.BlockSpec((_ROWS, HIDDEN), lambda i: (i, 0)),
            pl.BlockSpec((_ROWS, HIDDEN), lambda i: (i, 0)),
            pl.BlockSpec((_ROWS, HIDDEN), lambda i: (i, 0)),
        ],
        out_shape=[
            jax.ShapeDtypeStruct((N_PAD, HIDDEN), jnp.float32),
            jax.ShapeDtypeStruct((N_PAD, HIDDEN), jnp.float32),
            jax.ShapeDtypeStruct((N_PAD, HIDDEN), jnp.float32),
        ],
    )(x, p, u, bu, wc, bc)


def _last_body(x_ref, p_ref, u_ref, bu_ref, sc_ref, w_ref, b_ref, y_ref):
    x = x_ref[...]
    m = (p_ref[0] + p_ref[1]) * INV_SQRT_DEG
    h = jnp.dot(x + m, u_ref[...], preferred_element_type=jnp.float32)
    xn = x + jax.nn.silu(h + bu_ref[...])
    y_ref[...] = jnp.dot(sc_ref[...] + xn, w_ref[...],
                         preferred_element_type=jnp.float32) + b_ref[...]


def _last_call(x, p, u, bu, sc, w, b, odim):
    return pl.pallas_call(
        _last_body,
        grid=(_GRID,),
        in_specs=[
            pl.BlockSpec((_ROWS, HIDDEN), lambda i: (i, 0)),
            pl.BlockSpec((_NC, _ROWS, HIDDEN), lambda i: (0, i, 0)),
            pl.BlockSpec((HIDDEN, HIDDEN), lambda i: (0, 0)),
            pl.BlockSpec((1, HIDDEN), lambda i: (0, 0)),
            pl.BlockSpec((_ROWS, HIDDEN), lambda i: (i, 0)),
            pl.BlockSpec((HIDDEN, odim), lambda i: (0, 0)),
            pl.BlockSpec((1, odim), lambda i: (0, 0)),
        ],
        out_specs=pl.BlockSpec((_ROWS, odim), lambda i: (i, 0)),
        out_shape=jax.ShapeDtypeStruct((N_PAD, odim), jnp.float32),
    )(x, p, u, bu, sc, w, b)


def kernel(atomic_numbers_one_hot, pos, edge_index, emb_W, emb_b,
           interact_W, interact_b, update_W, update_b, out_W, out_b):
    n, h = N_NODES, HIDDEN
    feats = jnp.concatenate([atomic_numbers_one_hot, pos], axis=-1)
    fpad = h - feats.shape[1]
    feats = jnp.pad(feats, ((0, N_PAD - n), (0, fpad)))
    emb_Wp = jnp.pad(emb_W, ((0, fpad), (0, 0)))
    idx = jnp.stack([edge_index[0].reshape(_NW, _NCHUNK, _CHUNK),
                     edge_index[1].reshape(_NW, _NCHUNK, _CHUNK)], axis=2)

    wcat = [jnp.concatenate([interact_W[i, :h, :], interact_W[i, h:, :]], axis=1)
            for i in range(N_LAYERS)]
    bcat = [jnp.concatenate([jnp.zeros((h,), jnp.float32), interact_b[i]])[None, :]
            for i in range(N_LAYERS)]

    x, a, b = _embed_call(feats, emb_Wp, emb_b[None, :], wcat[0], bcat[0])
    sc = x
    for i in range(N_LAYERS):
        p = _make_edge_pass()(a, b, idx)
        if i < N_LAYERS - 1:
            x, a, b = _mid_call(x, p, update_W[i], update_b[i][None, :],
                                wcat[i + 1], bcat[i + 1])
        else:
            y = _last_call(x, p, update_W[i], update_b[i][None, :], sc,
                           out_W, out_b[None, :], out_W.shape[1])
    return y[:n]


# speedup vs baseline: 1.4874x; 1.4874x over previous
import functools
import math

import jax
import jax.numpy as jnp
from jax import lax
from jax.experimental import pallas as pl
from jax.experimental.pallas import tpu as pltpu
from jax.experimental.pallas import tpu_sc as plsc

N_NODES = 10000
N_PAD = 10240
N_EDGES = 320000
HIDDEN = 128
N_LAYERS = 4
INV_SQRT_DEG = 1.0 / math.sqrt(32.0)

_NC = 2
_NS = 16
_NW = _NC * _NS
_EPW = N_EDGES // _NW
_CHUNK = 40
_NCHUNK = _EPW // _CHUNK
_RPT = N_PAD // _NS


def _edge_body(a_hbm, b_hbm, idx_hbm, out_hbm,
               ib0, ib1, ib2, ib3, av0, av1, bv0, bv1, mv0, mv1, acc,
               si0, si1, si2, si3, sga0, sga1, sgb0, sgb1, ssc0, ssc1):
    c = lax.axis_index("c")
    s = lax.axis_index("s")
    wid = c * _NS + s
    ib = (ib0, ib1, ib2, ib3)
    sidx = (si0, si1, si2, si3)
    av = (av0, av1)
    bv = (bv0, bv1)
    mv = (mv0, mv1)
    sga = (sga0, sga1)
    sgb = (sgb0, sgb1)
    ssc = (ssc0, ssc1)

    for j in range(2):
        pltpu.async_copy(idx_hbm.at[wid, j], ib[j], sidx[j])

    zero16 = jnp.zeros((16,), jnp.float32)

    @plsc.parallel_loop(0, _CHUNK, step=1, unroll=2)
    def _zrow(i):
        for j in range(HIDDEN // 16):
            mv0[i, pl.ds(j * 16, 16)] = zero16
    for r in range(_RPT // _CHUNK):
        pltpu.sync_copy(mv0, acc.at[pl.ds(s * _RPT + r * _CHUNK, _CHUNK)])

    for j in range(2):
        pltpu.make_async_copy(idx_hbm.at[wid, j], ib[j], sidx[j]).wait()
        pltpu.async_copy(a_hbm.at[ib[j].at[0]], av[j], sga[j])
        pltpu.async_copy(b_hbm.at[ib[j].at[1]], bv[j], sgb[j])
    plsc.subcore_barrier()

    def _outer(g, carry):
        for b in range(4):
            k = 4 * g + b
            b2 = b % 2

            @pl.when(k < _NCHUNK)
            def _chunk():
                pltpu.make_async_copy(a_hbm.at[ib[b].at[0]], av[b2], sga[b2]).wait()
                pltpu.make_async_copy(b_hbm.at[ib[b].at[1]], bv[b2], sgb[b2]).wait()

                @pl.when(k >= 2)
                def _drain_prev_scatter():
                    pltpu.make_async_copy(
                        mv[b2], acc.at[ib[b].at[0]], ssc[b2]).wait()

                @pl.when(k + 2 < _NCHUNK)
                def _next_idx():
                    pltpu.async_copy(idx_hbm.at[wid, k + 2],
                                     ib[(b + 2) % 4], sidx[(b + 2) % 4])

                @plsc.parallel_loop(0, _CHUNK, step=1, unroll=1)
                def _row(i):
                    for j in range(HIDDEN // 16):
                        sl = pl.ds(j * 16, 16)
                        v = av[b2][i, sl] + bv[b2][i, sl]
                        mv[b2][i, sl] = v / (1.0 + jnp.exp(-v))

                @pl.when(k + 2 < _NCHUNK)
                def _next_gather():
                    pltpu.make_async_copy(idx_hbm.at[wid, k + 2],
                                          ib[(b + 2) % 4],
                                          sidx[(b + 2) % 4]).wait()
                    pltpu.async_copy(a_hbm.at[ib[(b + 2) % 4].at[0]],
                                     av[b2], sga[b2])
                    pltpu.async_copy(b_hbm.at[ib[(b + 2) % 4].at[1]],
                                     bv[b2], sgb[b2])

                pltpu.async_copy(mv[b2], acc.at[ib[b].at[0]], ssc[b2], add=True)

        return carry

    lax.fori_loop(0, (_NCHUNK + 3) // 4, _outer, 0)

    for b2 in range(2):
        pltpu.make_async_copy(mv[b2], acc.at[ib[0].at[0]], ssc[b2]).wait()

    plsc.subcore_barrier()
    off = s * _RPT
    pltpu.sync_copy(acc.at[pl.ds(off, _RPT)], out_hbm.at[c, pl.ds(off, _RPT)])


@functools.lru_cache(maxsize=None)
def _make_edge_pass():
    return functools.partial(
        pl.kernel,
        out_type=jax.ShapeDtypeStruct((_NC, N_PAD, HIDDEN), jnp.float32),
        mesh=plsc.VectorSubcoreMesh(core_axis_name="c", subcore_axis_name="s",
                                    num_cores=_NC, num_subcores=_NS),
        scratch_types=(
            [pltpu.VMEM((2, _CHUNK), jnp.int32) for _ in range(4)]
            + [pltpu.VMEM((_CHUNK, HIDDEN), jnp.float32) for _ in range(6)]
            + [pltpu.VMEM_SHARED((N_PAD, HIDDEN), jnp.float32)]
            + [pltpu.SemaphoreType.DMA for _ in range(10)]
        ),
    )(_edge_body)


_ROWS = 1280
_GRID = N_PAD // _ROWS


def _embed_body(f_ref, w_ref, b_ref, wc_ref, bc_ref, x_ref, a_ref, b2_ref):
    x = jnp.dot(f_ref[...], w_ref[...], preferred_element_type=jnp.float32)
    x = jax.nn.silu(x + b_ref[...])
    x_ref[...] = x
    ab = jnp.dot(x, wc_ref[...], preferred_element_type=jnp.float32) + bc_ref[...]
    a_ref[...] = ab[:, :HIDDEN]
    b2_ref[...] = ab[:, HIDDEN:]


def _embed_call(f, w, b, wc, bc):
    return pl.pallas_call(
        _embed_body,
        grid=(_GRID,),
        in_specs=[
            pl.BlockSpec((_ROWS, HIDDEN), lambda i: (i, 0)),
            pl.BlockSpec((HIDDEN, HIDDEN), lambda i: (0, 0)),
            pl.BlockSpec((1, HIDDEN), lambda i: (0, 0)),
            pl.BlockSpec((HIDDEN, 2 * HIDDEN), lambda i: (0, 0)),
            pl.BlockSpec((1, 2 * HIDDEN), lambda i: (0, 0)),
        ],
        out_specs=[
            pl.BlockSpec((_ROWS, HIDDEN), lambda i: (i, 0)),
            pl.BlockSpec((_ROWS, HIDDEN), lambda i: (i, 0)),
            pl.BlockSpec((_ROWS, HIDDEN), lambda i: (i, 0)),
        ],
        out_shape=[
            jax.ShapeDtypeStruct((N_PAD, HIDDEN), jnp.float32),
            jax.ShapeDtypeStruct((N_PAD, HIDDEN), jnp.float32),
            jax.ShapeDtypeStruct((N_PAD, HIDDEN), jnp.float32),
        ],
    )(f, w, b, wc, bc)


def _mid_body(x_ref, p_ref, u_ref, bu_ref, wc_ref, bc_ref, x_out, a_ref, b2_ref):
    x = x_ref[...]
    m = (p_ref[0] + p_ref[1]) * INV_SQRT_DEG
    h = jnp.dot(x + m, u_ref[...], preferred_element_type=jnp.float32)
    xn = x + jax.nn.silu(h + bu_ref[...])
    x_out[...] = xn
    ab = jnp.dot(xn, wc_ref[...], preferred_element_type=jnp.float32) + bc_ref[...]
    a_ref[...] = ab[:, :HIDDEN]
    b2_ref[...] = ab[:, HIDDEN:]


def _mid_call(x, p, u, bu, wc, bc):
    return pl.pallas_call(
        _mid_body,
        grid=(_GRID,),
        in_specs=[
            pl.BlockSpec((_ROWS, HIDDEN), lambda i: (i, 0)),
            pl.BlockSpec((_NC, _ROWS, HIDDEN), lambda i: (0, i, 0)),
            pl.BlockSpec((HIDDEN, HIDDEN), lambda i: (0, 0)),
            pl.BlockSpec((1, HIDDEN), lambda i: (0, 0)),
            pl.BlockSpec((HIDDEN, 2 * HIDDEN), lambda i: (0, 0)),
            pl.BlockSpec((1, 2 * HIDDEN), lambda i: (0, 0)),
        ],
        out_specs=[
            pl.BlockSpec((_ROWS, HIDDEN), lambda i: (i, 0)),
            pl.BlockSpec((_ROWS, HIDDEN), lambda i: (i, 0)),
            pl.BlockSpec((_ROWS, HIDDEN), lambda i: (i, 0)),
        ],
        out_shape=[
            jax.ShapeDtypeStruct((N_PAD, HIDDEN), jnp.float32),
            jax.ShapeDtypeStruct((N_PAD, HIDDEN), jnp.float32),
            jax.ShapeDtypeStruct((N_PAD, HIDDEN), jnp.float32),
        ],
    )(x, p, u, bu, wc, bc)


def _last_body(x_ref, p_ref, u_ref, bu_ref, sc_ref, w_ref, b_ref, y_ref):
    x = x_ref[...]
    m = (p_ref[0] + p_ref[1]) * INV_SQRT_DEG
    h = jnp.dot(x + m, u_ref[...], preferred_element_type=jnp.float32)
    xn = x + jax.nn.silu(h + bu_ref[...])
    y_ref[...] = jnp.dot(sc_ref[...] + xn, w_ref[...],
                         preferred_element_type=jnp.float32) + b_ref[...]


def _last_call(x, p, u, bu, sc, w, b, odim):
    return pl.pallas_call(
        _last_body,
        grid=(_GRID,),
        in_specs=[
            pl.BlockSpec((_ROWS, HIDDEN), lambda i: (i, 0)),
            pl.BlockSpec((_NC, _ROWS, HIDDEN), lambda i: (0, i, 0)),
            pl.BlockSpec((HIDDEN, HIDDEN), lambda i: (0, 0)),
            pl.BlockSpec((1, HIDDEN), lambda i: (0, 0)),
            pl.BlockSpec((_ROWS, HIDDEN), lambda i: (i, 0)),
            pl.BlockSpec((HIDDEN, odim), lambda i: (0, 0)),
            pl.BlockSpec((1, odim), lambda i: (0, 0)),
        ],
        out_specs=pl.BlockSpec((_ROWS, odim), lambda i: (i, 0)),
        out_shape=jax.ShapeDtypeStruct((N_PAD, odim), jnp.float32),
    )(x, p, u, bu, sc, w, b)


def kernel(atomic_numbers_one_hot, pos, edge_index, emb_W, emb_b,
           interact_W, interact_b, update_W, update_b, out_W, out_b):
    n, h = N_NODES, HIDDEN
    feats = jnp.concatenate([atomic_numbers_one_hot, pos], axis=-1)
    fpad = h - feats.shape[1]
    feats = jnp.pad(feats, ((0, N_PAD - n), (0, fpad)))
    emb_Wp = jnp.pad(emb_W, ((0, fpad), (0, 0)))
    idx = jnp.stack([edge_index[0].reshape(_NW, _NCHUNK, _CHUNK),
                     edge_index[1].reshape(_NW, _NCHUNK, _CHUNK)], axis=2)

    wcat = [jnp.concatenate([interact_W[i, :h, :], interact_W[i, h:, :]], axis=1)
            for i in range(N_LAYERS)]
    bcat = [jnp.concatenate([jnp.zeros((h,), jnp.float32), interact_b[i]])[None, :]
            for i in range(N_LAYERS)]

    x, a, b = _embed_call(feats, emb_Wp, emb_b[None, :], wcat[0], bcat[0])
    sc = x
    for i in range(N_LAYERS):
        p = _make_edge_pass()(a, b, idx)
        if i < N_LAYERS - 1:
            x, a, b = _mid_call(x, p, update_W[i], update_b[i][None, :],
                                wcat[i + 1], bcat[i + 1])
        else:
            y = _last_call(x, p, update_W[i], update_b[i][None, :], sc,
                           out_W, out_b[None, :], out_W.shape[1])
    return y[:n]


# speedup vs baseline: 1.5571x; 1.0469x over previous
import functools
import math

import jax
import jax.numpy as jnp
from jax import lax
from jax.experimental import pallas as pl
from jax.experimental.pallas import tpu as pltpu
from jax.experimental.pallas import tpu_sc as plsc

N_NODES = 10000
N_PAD = 10240
N_EDGES = 320000
HIDDEN = 128
N_LAYERS = 4
INV_SQRT_DEG = 1.0 / math.sqrt(32.0)

_NC = 2
_NS = 16
_NW = _NC * _NS
_EPW = N_EDGES // _NW
_CHUNK = 40
_NCHUNK = _EPW // _CHUNK
_RPT = N_PAD // _NS


def _edge_body(a_hbm, b_hbm, idx_hbm, out_hbm,
               ib0, ib1, ib2, ib3, av0, av1, bv0, bv1, mv0, mv1, acc,
               si0, si1, si2, si3, sga0, sga1, sgb0, sgb1, ssc0, ssc1):
    c = lax.axis_index("c")
    s = lax.axis_index("s")
    wid = c * _NS + s
    ib = (ib0, ib1, ib2, ib3)
    sidx = (si0, si1, si2, si3)
    av = (av0, av1)
    bv = (bv0, bv1)
    mv = (mv0, mv1)
    sga = (sga0, sga1)
    sgb = (sgb0, sgb1)
    ssc = (ssc0, ssc1)

    for j in range(2):
        pltpu.async_copy(idx_hbm.at[wid, j], ib[j], sidx[j])

    zero16 = jnp.zeros((16,), jnp.float32)

    @plsc.parallel_loop(0, _CHUNK, step=1, unroll=2)
    def _zrow(i):
        for j in range(HIDDEN // 16):
            mv0[i, pl.ds(j * 16, 16)] = zero16
    for r in range(_RPT // _CHUNK):
        pltpu.sync_copy(mv0, acc.at[pl.ds(s * _RPT + r * _CHUNK, _CHUNK)])

    for j in range(2):
        pltpu.make_async_copy(idx_hbm.at[wid, j], ib[j], sidx[j]).wait()
        pltpu.async_copy(a_hbm.at[ib[j].at[0]], av[j], sga[j])
        pltpu.async_copy(b_hbm.at[ib[j].at[1]], bv[j], sgb[j])
    plsc.subcore_barrier()

    def _outer(g, carry):
        for b in range(4):
            k = 4 * g + b
            b2 = b % 2

            @pl.when(k < _NCHUNK)
            def _chunk():
                @pl.when(k >= 2)
                def _drain_prev_scatter():
                    pltpu.make_async_copy(
                        mv[b2], acc.at[ib[b].at[0]], ssc[b2]).wait()

                @pl.when(k + 2 < _NCHUNK)
                def _next_idx():
                    pltpu.async_copy(idx_hbm.at[wid, k + 2],
                                     ib[(b + 2) % 4], sidx[(b + 2) % 4])

                pltpu.make_async_copy(a_hbm.at[ib[b].at[0]], av[b2], sga[b2]).wait()
                pltpu.make_async_copy(b_hbm.at[ib[b].at[1]], bv[b2], sgb[b2]).wait()

                @plsc.parallel_loop(0, _CHUNK, step=1, unroll=1)
                def _row(i):
                    for j in range(HIDDEN // 16):
                        sl = pl.ds(j * 16, 16)
                        v = av[b2][i, sl] + bv[b2][i, sl]
                        mv[b2][i, sl] = v / (1.0 + jnp.exp(-v))

                @pl.when(k + 2 < _NCHUNK)
                def _next_gather():
                    pltpu.make_async_copy(idx_hbm.at[wid, k + 2],
                                          ib[(b + 2) % 4],
                                          sidx[(b + 2) % 4]).wait()
                    pltpu.async_copy(a_hbm.at[ib[(b + 2) % 4].at[0]],
                                     av[b2], sga[b2])
                    pltpu.async_copy(b_hbm.at[ib[(b + 2) % 4].at[1]],
                                     bv[b2], sgb[b2])

                pltpu.async_copy(mv[b2], acc.at[ib[b].at[0]], ssc[b2], add=True)

        return carry

    lax.fori_loop(0, (_NCHUNK + 3) // 4, _outer, 0)

    for b2 in range(2):
        pltpu.make_async_copy(mv[b2], acc.at[ib[0].at[0]], ssc[b2]).wait()

    plsc.subcore_barrier()
    off = s * _RPT
    pltpu.sync_copy(acc.at[pl.ds(off, _RPT)], out_hbm.at[c, pl.ds(off, _RPT)])


@functools.lru_cache(maxsize=None)
def _make_edge_pass():
    return functools.partial(
        pl.kernel,
        out_type=jax.ShapeDtypeStruct((_NC, N_PAD, HIDDEN), jnp.float32),
        mesh=plsc.VectorSubcoreMesh(core_axis_name="c", subcore_axis_name="s",
                                    num_cores=_NC, num_subcores=_NS),
        scratch_types=(
            [pltpu.VMEM((2, _CHUNK), jnp.int32) for _ in range(4)]
            + [pltpu.VMEM((_CHUNK, HIDDEN), jnp.float32) for _ in range(6)]
            + [pltpu.VMEM_SHARED((N_PAD, HIDDEN), jnp.float32)]
            + [pltpu.SemaphoreType.DMA for _ in range(10)]
        ),
    )(_edge_body)


_ROWS = 1280
_GRID = N_PAD // _ROWS


def _embed_body(f_ref, w_ref, b_ref, wc_ref, bc_ref, x_ref, a_ref, b2_ref):
    x = jnp.dot(f_ref[...], w_ref[...], preferred_element_type=jnp.float32)
    x = jax.nn.silu(x + b_ref[...])
    x_ref[...] = x
    ab = jnp.dot(x, wc_ref[...], preferred_element_type=jnp.float32) + bc_ref[...]
    a_ref[...] = ab[:, :HIDDEN]
    b2_ref[...] = ab[:, HIDDEN:]


def _embed_call(f, w, b, wc, bc):
    return pl.pallas_call(
        _embed_body,
        grid=(_GRID,),
        in_specs=[
            pl.BlockSpec((_ROWS, HIDDEN), lambda i: (i, 0)),
            pl.BlockSpec((HIDDEN, HIDDEN), lambda i: (0, 0)),
            pl.BlockSpec((1, HIDDEN), lambda i: (0, 0)),
            pl.BlockSpec((HIDDEN, 2 * HIDDEN), lambda i: (0, 0)),
            pl.BlockSpec((1, 2 * HIDDEN), lambda i: (0, 0)),
        ],
        out_specs=[
            pl.BlockSpec((_ROWS, HIDDEN), lambda i: (i, 0)),
            pl.BlockSpec((_ROWS, HIDDEN), lambda i: (i, 0)),
            pl.BlockSpec((_ROWS, HIDDEN), lambda i: (i, 0)),
        ],
        out_shape=[
            jax.ShapeDtypeStruct((N_PAD, HIDDEN), jnp.float32),
            jax.ShapeDtypeStruct((N_PAD, HIDDEN), jnp.float32),
            jax.ShapeDtypeStruct((N_PAD, HIDDEN), jnp.float32),
        ],
    )(f, w, b, wc, bc)


def _mid_body(x_ref, p_ref, u_ref, bu_ref, wc_ref, bc_ref, x_out, a_ref, b2_ref):
    x = x_ref[...]
    m = (p_ref[0] + p_ref[1]) * INV_SQRT_DEG
    h = jnp.dot(x + m, u_ref[...], preferred_element_type=jnp.float32)
    xn = x + jax.nn.silu(h + bu_ref[...])
    x_out[...] = xn
    ab = jnp.dot(xn, wc_ref[...], preferred_element_type=jnp.float32) + bc_ref[...]
    a_ref[...] = ab[:, :HIDDEN]
    b2_ref[...] = ab[:, HIDDEN:]


def _mid_call(x, p, u, bu, wc, bc):
    return pl.pallas_call(
        _mid_body,
        grid=(_GRID,),
        in_specs=[
            pl.BlockSpec((_ROWS, HIDDEN), lambda i: (i, 0)),
            pl.BlockSpec((_NC, _ROWS, HIDDEN), lambda i: (0, i, 0)),
            pl.BlockSpec((HIDDEN, HIDDEN), lambda i: (0, 0)),
            pl.BlockSpec((1, HIDDEN), lambda i: (0, 0)),
            pl.BlockSpec((HIDDEN, 2 * HIDDEN), lambda i: (0, 0)),
            pl.BlockSpec((1, 2 * HIDDEN), lambda i: (0, 0)),
        ],
        out_specs=[
            pl.BlockSpec((_ROWS, HIDDEN), lambda i: (i, 0)),
            pl.BlockSpec((_ROWS, HIDDEN), lambda i: (i, 0)),
            pl.BlockSpec((_ROWS, HIDDEN), lambda i: (i, 0)),
        ],
        out_shape=[
            jax.ShapeDtypeStruct((N_PAD, HIDDEN), jnp.float32),
            jax.ShapeDtypeStruct((N_PAD, HIDDEN), jnp.float32),
            jax.ShapeDtypeStruct((N_PAD, HIDDEN), jnp.float32),
        ],
    )(x, p, u, bu, wc, bc)


def _last_body(x_ref, p_ref, u_ref, bu_ref, sc_ref, w_ref, b_ref, y_ref):
    x = x_ref[...]
    m = (p_ref[0] + p_ref[1]) * INV_SQRT_DEG
    h = jnp.dot(x + m, u_ref[...], preferred_element_type=jnp.float32)
    xn = x + jax.nn.silu(h + bu_ref[...])
    y_ref[...] = jnp.dot(sc_ref[...] + xn, w_ref[...],
                         preferred_element_type=jnp.float32) + b_ref[...]


def _last_call(x, p, u, bu, sc, w, b, odim):
    return pl.pallas_call(
        _last_body,
        grid=(_GRID,),
        in_specs=[
            pl.BlockSpec((_ROWS, HIDDEN), lambda i: (i, 0)),
            pl.BlockSpec((_NC, _ROWS, HIDDEN), lambda i: (0, i, 0)),
            pl.BlockSpec((HIDDEN, HIDDEN), lambda i: (0, 0)),
            pl.BlockSpec((1, HIDDEN), lambda i: (0, 0)),
            pl.BlockSpec((_ROWS, HIDDEN), lambda i: (i, 0)),
            pl.BlockSpec((HIDDEN, odim), lambda i: (0, 0)),
            pl.BlockSpec((1, odim), lambda i: (0, 0)),
        ],
        out_specs=pl.BlockSpec((_ROWS, odim), lambda i: (i, 0)),
        out_shape=jax.ShapeDtypeStruct((N_PAD, odim), jnp.float32),
    )(x, p, u, bu, sc, w, b)


def kernel(atomic_numbers_one_hot, pos, edge_index, emb_W, emb_b,
           interact_W, interact_b, update_W, update_b, out_W, out_b):
    n, h = N_NODES, HIDDEN
    feats = jnp.concatenate([atomic_numbers_one_hot, pos], axis=-1)
    fpad = h - feats.shape[1]
    feats = jnp.pad(feats, ((0, N_PAD - n), (0, fpad)))
    emb_Wp = jnp.pad(emb_W, ((0, fpad), (0, 0)))
    idx = jnp.stack([edge_index[0].reshape(_NW, _NCHUNK, _CHUNK),
                     edge_index[1].reshape(_NW, _NCHUNK, _CHUNK)], axis=2)

    wcat = [jnp.concatenate([interact_W[i, :h, :], interact_W[i, h:, :]], axis=1)
            for i in range(N_LAYERS)]
    bcat = [jnp.concatenate([jnp.zeros((h,), jnp.float32), interact_b[i]])[None, :]
            for i in range(N_LAYERS)]

    x, a, b = _embed_call(feats, emb_Wp, emb_b[None, :], wcat[0], bcat[0])
    sc = x
    for i in range(N_LAYERS):
        p = _make_edge_pass()(a, b, idx)
        if i < N_LAYERS - 1:
            x, a, b = _mid_call(x, p, update_W[i], update_b[i][None, :],
                                wcat[i + 1], bcat[i + 1])
        else:
            y = _last_call(x, p, update_W[i], update_b[i][None, :], sc,
                           out_W, out_b[None, :], out_W.shape[1])
    return y[:n]


# speedup vs baseline: 1.5602x; 1.0019x over previous
import functools
import math

import jax
import jax.numpy as jnp
from jax import lax
from jax.experimental import pallas as pl
from jax.experimental.pallas import tpu as pltpu
from jax.experimental.pallas import tpu_sc as plsc

N_NODES = 10000
N_PAD = 10240
N_EDGES = 320000
HIDDEN = 128
N_LAYERS = 4
INV_SQRT_DEG = 1.0 / math.sqrt(32.0)

_NC = 2
_NS = 16
_NW = _NC * _NS
_EPW = N_EDGES // _NW
_CHUNK = 40
_NCHUNK = _EPW // _CHUNK
_RPT = N_PAD // _NS


def _edge_body(a_hbm, b_hbm, idx_hbm, out_hbm,
               ib0, ib1, ib2, ib3, av0, av1, bv0, bv1, mv0, mv1, acc,
               si0, si1, si2, si3, sga0, sga1, sgb0, sgb1, ssc0, ssc1):
    c = lax.axis_index("c")
    s = lax.axis_index("s")
    wid = c * _NS + s
    ib = (ib0, ib1, ib2, ib3)
    sidx = (si0, si1, si2, si3)
    av = (av0, av1)
    bv = (bv0, bv1)
    mv = (mv0, mv1)
    sga = (sga0, sga1)
    sgb = (sgb0, sgb1)
    ssc = (ssc0, ssc1)

    for j in range(2):
        pltpu.async_copy(idx_hbm.at[wid, j], ib[j], sidx[j])

    zero16 = jnp.zeros((16,), jnp.float32)

    @plsc.parallel_loop(0, _CHUNK, step=1, unroll=2)
    def _zrow(i):
        for j in range(HIDDEN // 16):
            mv0[i, pl.ds(j * 16, 16)] = zero16
    for r in range(_RPT // _CHUNK):
        pltpu.sync_copy(mv0, acc.at[pl.ds(s * _RPT + r * _CHUNK, _CHUNK)])

    for j in range(2):
        pltpu.make_async_copy(idx_hbm.at[wid, j], ib[j], sidx[j]).wait()
        pltpu.async_copy(a_hbm.at[ib[j].at[0]], av[j], sga[j])
        pltpu.async_copy(b_hbm.at[ib[j].at[1]], bv[j], sgb[j])
    plsc.subcore_barrier()

    def _outer(g, carry):
        for b in range(4):
            k = 4 * g + b
            b2 = b % 2

            @pl.when(k < _NCHUNK)
            def _chunk():
                @pl.when(k >= 2)
                def _drain_prev_scatter():
                    pltpu.make_async_copy(
                        mv[b2], acc.at[ib[b].at[0]], ssc[b2]).wait()

                @pl.when(k + 2 < _NCHUNK)
                def _next_idx():
                    pltpu.async_copy(idx_hbm.at[wid, k + 2],
                                     ib[(b + 2) % 4], sidx[(b + 2) % 4])

                pltpu.make_async_copy(a_hbm.at[ib[b].at[0]], av[b2], sga[b2]).wait()
                pltpu.make_async_copy(b_hbm.at[ib[b].at[1]], bv[b2], sgb[b2]).wait()

                @plsc.parallel_loop(0, _CHUNK, step=1, unroll=1)
                def _row(i):
                    for j in range(HIDDEN // 16):
                        sl = pl.ds(j * 16, 16)
                        v = av[b2][i, sl] + bv[b2][i, sl]
                        mv[b2][i, sl] = v / (1.0 + jnp.exp(-v))

                pltpu.async_copy(mv[b2], acc.at[ib[b].at[0]], ssc[b2], add=True)

                @pl.when(k + 2 < _NCHUNK)
                def _next_gather():
                    pltpu.make_async_copy(idx_hbm.at[wid, k + 2],
                                          ib[(b + 2) % 4],
                                          sidx[(b + 2) % 4]).wait()
                    pltpu.async_copy(a_hbm.at[ib[(b + 2) % 4].at[0]],
                                     av[b2], sga[b2])
                    pltpu.async_copy(b_hbm.at[ib[(b + 2) % 4].at[1]],
                                     bv[b2], sgb[b2])

        return carry

    lax.fori_loop(0, (_NCHUNK + 3) // 4, _outer, 0)

    for b2 in range(2):
        pltpu.make_async_copy(mv[b2], acc.at[ib[0].at[0]], ssc[b2]).wait()

    plsc.subcore_barrier()
    off = s * _RPT
    pltpu.sync_copy(acc.at[pl.ds(off, _RPT)], out_hbm.at[c, pl.ds(off, _RPT)])


@functools.lru_cache(maxsize=None)
def _make_edge_pass():
    return functools.partial(
        pl.kernel,
        out_type=jax.ShapeDtypeStruct((_NC, N_PAD, HIDDEN), jnp.float32),
        mesh=plsc.VectorSubcoreMesh(core_axis_name="c", subcore_axis_name="s",
                                    num_cores=_NC, num_subcores=_NS),
        scratch_types=(
            [pltpu.VMEM((2, _CHUNK), jnp.int32) for _ in range(4)]
            + [pltpu.VMEM((_CHUNK, HIDDEN), jnp.float32) for _ in range(6)]
            + [pltpu.VMEM_SHARED((N_PAD, HIDDEN), jnp.float32)]
            + [pltpu.SemaphoreType.DMA for _ in range(10)]
        ),
    )(_edge_body)


_ROWS = 1280
_GRID = N_PAD // _ROWS


def _embed_body(f_ref, w_ref, b_ref, wc_ref, bc_ref, x_ref, a_ref, b2_ref):
    x = jnp.dot(f_ref[...], w_ref[...], preferred_element_type=jnp.float32)
    x = jax.nn.silu(x + b_ref[...])
    x_ref[...] = x
    ab = jnp.dot(x, wc_ref[...], preferred_element_type=jnp.float32) + bc_ref[...]
    a_ref[...] = ab[:, :HIDDEN]
    b2_ref[...] = ab[:, HIDDEN:]


def _embed_call(f, w, b, wc, bc):
    return pl.pallas_call(
        _embed_body,
        grid=(_GRID,),
        in_specs=[
            pl.BlockSpec((_ROWS, HIDDEN), lambda i: (i, 0)),
            pl.BlockSpec((HIDDEN, HIDDEN), lambda i: (0, 0)),
            pl.BlockSpec((1, HIDDEN), lambda i: (0, 0)),
            pl.BlockSpec((HIDDEN, 2 * HIDDEN), lambda i: (0, 0)),
            pl.BlockSpec((1, 2 * HIDDEN), lambda i: (0, 0)),
        ],
        out_specs=[
            pl.BlockSpec((_ROWS, HIDDEN), lambda i: (i, 0)),
            pl.BlockSpec((_ROWS, HIDDEN), lambda i: (i, 0)),
            pl.BlockSpec((_ROWS, HIDDEN), lambda i: (i, 0)),
        ],
        out_shape=[
            jax.ShapeDtypeStruct((N_PAD, HIDDEN), jnp.float32),
            jax.ShapeDtypeStruct((N_PAD, HIDDEN), jnp.float32),
            jax.ShapeDtypeStruct((N_PAD, HIDDEN), jnp.float32),
        ],
    )(f, w, b, wc, bc)


def _mid_body(x_ref, p_ref, u_ref, bu_ref, wc_ref, bc_ref, x_out, a_ref, b2_ref):
    x = x_ref[...]
    m = (p_ref[0] + p_ref[1]) * INV_SQRT_DEG
    h = jnp.dot(x + m, u_ref[...], preferred_element_type=jnp.float32)
    xn = x + jax.nn.silu(h + bu_ref[...])
    x_out[...] = xn
    ab = jnp.dot(xn, wc_ref[...], preferred_element_type=jnp.float32) + bc_ref[...]
    a_ref[...] = ab[:, :HIDDEN]
    b2_ref[...] = ab[:, HIDDEN:]


def _mid_call(x, p, u, bu, wc, bc):
    return pl.pallas_call(
        _mid_body,
        grid=(_GRID,),
        in_specs=[
            pl.BlockSpec((_ROWS, HIDDEN), lambda i: (i, 0)),
            pl.BlockSpec((_NC, _ROWS, HIDDEN), lambda i: (0, i, 0)),
            pl.BlockSpec((HIDDEN, HIDDEN), lambda i: (0, 0)),
            pl.BlockSpec((1, HIDDEN), lambda i: (0, 0)),
            pl.BlockSpec((HIDDEN, 2 * HIDDEN), lambda i: (0, 0)),
            pl.BlockSpec((1, 2 * HIDDEN), lambda i: (0, 0)),
        ],
        out_specs=[
            pl.BlockSpec((_ROWS, HIDDEN), lambda i: (i, 0)),
            pl.BlockSpec((_ROWS, HIDDEN), lambda i: (i, 0)),
            pl.BlockSpec((_ROWS, HIDDEN), lambda i: (i, 0)),
        ],
        out_shape=[
            jax.ShapeDtypeStruct((N_PAD, HIDDEN), jnp.float32),
            jax.ShapeDtypeStruct((N_PAD, HIDDEN), jnp.float32),
            jax.ShapeDtypeStruct((N_PAD, HIDDEN), jnp.float32),
        ],
    )(x, p, u, bu, wc, bc)


def _last_body(x_ref, p_ref, u_ref, bu_ref, sc_ref, w_ref, b_ref, y_ref):
    x = x_ref[...]
    m = (p_ref[0] + p_ref[1]) * INV_SQRT_DEG
    h = jnp.dot(x + m, u_ref[...], preferred_element_type=jnp.float32)
    xn = x + jax.nn.silu(h + bu_ref[...])
    y_ref[...] = jnp.dot(sc_ref[...] + xn, w_ref[...],
                         preferred_element_type=jnp.float32) + b_ref[...]


def _last_call(x, p, u, bu, sc, w, b, odim):
    return pl.pallas_call(
        _last_body,
        grid=(_GRID,),
        in_specs=[
            pl.BlockSpec((_ROWS, HIDDEN), lambda i: (i, 0)),
            pl.BlockSpec((_NC, _ROWS, HIDDEN), lambda i: (0, i, 0)),
            pl.BlockSpec((HIDDEN, HIDDEN), lambda i: (0, 0)),
            pl.BlockSpec((1, HIDDEN), lambda i: (0, 0)),
            pl.BlockSpec((_ROWS, HIDDEN), lambda i: (i, 0)),
            pl.BlockSpec((HIDDEN, odim), lambda i: (0, 0)),
            pl.BlockSpec((1, odim), lambda i: (0, 0)),
        ],
        out_specs=pl.BlockSpec((_ROWS, odim), lambda i: (i, 0)),
        out_shape=jax.ShapeDtypeStruct((N_PAD, odim), jnp.float32),
    )(x, p, u, bu, sc, w, b)


def kernel(atomic_numbers_one_hot, pos, edge_index, emb_W, emb_b,
           interact_W, interact_b, update_W, update_b, out_W, out_b):
    n, h = N_NODES, HIDDEN
    feats = jnp.concatenate([atomic_numbers_one_hot, pos], axis=-1)
    fpad = h - feats.shape[1]
    feats = jnp.pad(feats, ((0, N_PAD - n), (0, fpad)))
    emb_Wp = jnp.pad(emb_W, ((0, fpad), (0, 0)))
    idx = jnp.stack([edge_index[0].reshape(_NW, _NCHUNK, _CHUNK),
                     edge_index[1].reshape(_NW, _NCHUNK, _CHUNK)], axis=2)

    wcat = [jnp.concatenate([interact_W[i, :h, :], interact_W[i, h:, :]], axis=1)
            for i in range(N_LAYERS)]
    bcat = [jnp.concatenate([jnp.zeros((h,), jnp.float32), interact_b[i]])[None, :]
            for i in range(N_LAYERS)]

    x, a, b = _embed_call(feats, emb_Wp, emb_b[None, :], wcat[0], bcat[0])
    sc = x
    for i in range(N_LAYERS):
        p = _make_edge_pass()(a, b, idx)
        if i < N_LAYERS - 1:
            x, a, b = _mid_call(x, p, update_W[i], update_b[i][None, :],
                                wcat[i + 1], bcat[i + 1])
        else:
            y = _last_call(x, p, update_W[i], update_b[i][None, :], sc,
                           out_W, out_b[None, :], out_W.shape[1])
    return y[:n]
